# baseline (speedup 1.0000x reference)
import functools

import jax
import jax.numpy as jnp
import numpy as np
from jax import lax
from jax.experimental import pallas as pl
from jax.experimental.pallas import tpu as pltpu
from jax.experimental.pallas import tpu_sc as plsc

KEEP = 32
NW = 32
R_BLK = 32
ROWS_PER_W = 512
N_BLK = ROWS_PER_W // R_BLK


def _resize_matrix(out_n: int, in_n: int) -> np.ndarray:
    scale = out_n / in_n
    kernel_scale = max(1.0, 1.0 / scale)
    sample_f = (np.arange(out_n) + 0.5) / scale - 0.5
    x = np.abs(sample_f[:, None] - np.arange(in_n)[None, :]) / kernel_scale
    a = np.maximum(0.0, 1.0 - x)
    a = a / a.sum(axis=1, keepdims=True)
    return a.astype(np.float32)


_A = _resize_matrix(32, 512)


MB = 4


def _mask_body(a_ref, at_ref, m_ref, o_ref):
    for i in range(MB):
        t = jnp.dot(a_ref[...], m_ref[i], preferred_element_type=jnp.float32,
                    precision=jax.lax.Precision.HIGHEST)
        s = jnp.dot(t, at_ref[...], preferred_element_type=jnp.float32,
                    precision=jax.lax.Precision.HIGHEST)
        o_ref[i] = (s > 0.5).astype(jnp.float32)


def _compute_fg(ref_mask_sq):
    b = ref_mask_sq.shape[0]
    a = jnp.asarray(_A)
    return pl.pallas_call(
        _mask_body,
        grid=(b // MB,),
        in_specs=[
            pl.BlockSpec((32, 512), lambda i: (0, 0)),
            pl.BlockSpec((512, 32), lambda i: (0, 0)),
            pl.BlockSpec((MB, 512, 512), lambda i: (i, 0, 0)),
        ],
        out_specs=pl.BlockSpec((MB, 32, 32), lambda i: (i, 0, 0)),
        out_shape=jax.ShapeDtypeStruct((b, 32, 32), jnp.float32),
    )(a, a.T, ref_mask_sq)


def _sort_a(x):
    return plsc.sort_key_val(x, x)[0]


def _sort_d(x):
    return plsc.sort_key_val(x, x, descending=True)[0]


def _update(r0d, r1d, c0, c1):
    c0s = _sort_a(c0)
    c1d = _sort_d(c1)
    lo = jnp.minimum(c0s, c1d)
    hi = jnp.maximum(c0s, c1d)
    s0 = _sort_a(lo)
    s1 = _sort_a(hi)
    h0 = jnp.maximum(s0, r1d)
    h1 = jnp.maximum(s1, r0d)
    lo2 = jnp.minimum(h0, h1)
    hi2 = jnp.maximum(h0, h1)
    return _sort_d(lo2), _sort_d(hi2)


def _merge32(r0d, r1d, q0d, q1d):
    s0 = jnp.flip(q0d, 0)
    s1 = jnp.flip(q1d, 0)
    h0 = jnp.maximum(s0, r1d)
    h1 = jnp.maximum(s1, r0d)
    lo2 = jnp.minimum(h0, h1)
    hi2 = jnp.maximum(h0, h1)
    return _sort_d(lo2), _sort_d(hi2)


def _topk_body(corr_hbm, fg_hbm, out_hbm, mask_v, buf, res, candf, candg,
               candf2, candg2, sem0, sem1):
    w = lax.axis_index("s") * 2 + lax.axis_index("c")
    b = w // 2
    half = w % 2
    row0 = half * ROWS_PER_W

    ks = lax.iota(jnp.int32, 16)
    neg = jnp.full((16,), -jnp.inf, jnp.float32)

    def start(gb, slot, sem):
        pltpu.make_async_copy(
            corr_hbm.at[b, pl.ds(row0 + gb * R_BLK, R_BLK), :],
            buf.at[slot], sem).start()

    def wait(slot, sem):
        pltpu.make_async_copy(
            corr_hbm.at[b, pl.ds(row0, R_BLK), :], buf.at[slot], sem).wait()

    N_STATIC = 4

    def consume(cand, offs):
        def body(j, st):
            base = j * 32
            c0 = cand[pl.ds(base, 16)]
            c1 = cand[pl.ds(base + 16, 16)]
            j2 = jnp.full((16,), 2 * j, jnp.int32)
            c0 = jnp.where(j2 < offs, c0, neg)
            c1 = jnp.where(j2 + 1 < offs, c1, neg)
            return _update(st[0], st[1], c0, c1)

        trips = (jnp.max(offs) + 1) // 2
        return lax.fori_loop(0, trips, body, (neg, neg))

    def do_rows(gb, slot):
        def row_body(i, _):
            r = 2 * i
            p = gb * R_BLK + r
            fm1a, fm2a, gm1a, gm2a = neg, neg, neg, neg
            fm1b, fm2b, gm1b, gm2b = neg, neg, neg, neg
            for k in range(64):
                off = k * 16
                m = mask_v[pl.ds(off, 16)]
                va = buf[slot, r, pl.ds(off, 16)]
                vb = buf[slot, r + 1, pl.ds(off, 16)]
                aa = va * m
                da = va - aa
                ab = vb * m
                db = vb - ab
                lo = jnp.minimum(aa, fm1a)
                fm1a = jnp.maximum(aa, fm1a)
                fm2a = jnp.maximum(fm2a, lo)
                lo = jnp.minimum(da, gm1a)
                gm1a = jnp.maximum(da, gm1a)
                gm2a = jnp.maximum(gm2a, lo)
                lo = jnp.minimum(ab, fm1b)
                fm1b = jnp.maximum(ab, fm1b)
                fm2b = jnp.maximum(fm2b, lo)
                lo = jnp.minimum(db, gm1b)
                gm1b = jnp.maximum(db, gm1b)
                gm2b = jnp.maximum(gm2b, lo)
            tfva = jnp.full((16,), jnp.min(fm2a), jnp.float32)
            tgva = jnp.full((16,), jnp.min(gm2a), jnp.float32)
            tfvb = jnp.full((16,), jnp.min(fm2b), jnp.float32)
            tgvb = jnp.full((16,), jnp.min(gm2b), jnp.float32)
            idxfa, idxga, idxfb, idxgb = ks, ks, ks, ks
            for k in range(64):
                off = k * 16
                m = mask_v[pl.ds(off, 16)]
                va = buf[slot, r, pl.ds(off, 16)]
                vb = buf[slot, r + 1, pl.ds(off, 16)]
                aa = va * m
                da = va - aa
                ab = vb * m
                db = vb - ab
                sfa = aa >= tfva
                sga = da >= tgva
                sfb = ab >= tfvb
                sgb = db >= tgvb
                plsc.store_scatter(candf, [idxfa], aa, mask=sfa)
                plsc.store_scatter(candg, [idxga], da, mask=sga)
                plsc.store_scatter(candf2, [idxfb], ab, mask=sfb)
                plsc.store_scatter(candg2, [idxgb], db, mask=sgb)
                idxfa = idxfa + (sfa.astype(jnp.int32) << 4)
                idxga = idxga + (sga.astype(jnp.int32) << 4)
                idxfb = idxfb + (sfb.astype(jnp.int32) << 4)
                idxgb = idxgb + (sgb.astype(jnp.int32) << 4)
            f0a, f1a = consume(candf, (idxfa - ks) >> 4)
            g0a, g1a = consume(candg, (idxga - ks) >> 4)
            f0b, f1b = consume(candf2, (idxfb - ks) >> 4)
            g0b, g1b = consume(candg2, (idxgb - ks) >> 4)
            pv = jnp.full((16,), p, jnp.int32)
            plsc.store_scatter(res, [ks, pv], g1a)
            plsc.store_scatter(res, [ks + 16, pv], g0a)
            plsc.store_scatter(res, [ks + 32, pv], f1a)
            plsc.store_scatter(res, [ks + 48, pv], f0a)
            pv = pv + 1
            plsc.store_scatter(res, [ks, pv], g1b)
            plsc.store_scatter(res, [ks + 16, pv], g0b)
            plsc.store_scatter(res, [ks + 32, pv], f1b)
            plsc.store_scatter(res, [ks + 48, pv], f0b)
            return 0

        lax.fori_loop(0, R_BLK // 2, row_body, 0)

    start(0, 0, sem0)
    pltpu.sync_copy(fg_hbm.at[b], mask_v)

    def block_body(g2, _):
        gb0 = 2 * g2
        wait(0, sem0)
        start(gb0 + 1, 1, sem1)
        do_rows(gb0, 0)
        wait(1, sem1)

        @pl.when(g2 < N_BLK // 2 - 1)
        def _():
            start(gb0 + 2, 0, sem0)

        do_rows(gb0 + 1, 1)
        return 0

    lax.fori_loop(0, N_BLK // 2, block_body, 0)
    pltpu.sync_copy(res, out_hbm.at[b, :, pl.ds(row0, ROWS_PER_W)])


def _masked_topk(corr3, fg_flat):
    mesh = plsc.VectorSubcoreMesh(core_axis_name="c", subcore_axis_name="s",
                                  num_cores=2, num_subcores=16)
    f = pl.kernel(
        _topk_body,
        out_type=jax.ShapeDtypeStruct((16, 2 * KEEP, 1024), jnp.float32),
        mesh=mesh,
        compiler_params=pltpu.CompilerParams(needs_layout_passes=False),
        scratch_types=[
            pltpu.VMEM((1024,), jnp.float32),
            pltpu.VMEM((2, R_BLK, 1024), jnp.float32),
            pltpu.VMEM((2 * KEEP, ROWS_PER_W), jnp.float32),
            pltpu.VMEM((1040,), jnp.float32),
            pltpu.VMEM((1040,), jnp.float32),
            pltpu.VMEM((1040,), jnp.float32),
            pltpu.VMEM((1040,), jnp.float32),
            pltpu.SemaphoreType.DMA,
            pltpu.SemaphoreType.DMA,
        ],
    )
    return f(corr3, fg_flat)


def kernel(corr_features, ref_mask):
    batch, cur_h, cur_w, ref_h, ref_w = corr_features.shape
    corr3 = corr_features.reshape(batch, cur_h * cur_w, ref_h * ref_w)
    fg = _compute_fg(ref_mask.reshape(batch, 512, 512))
    fg_flat = fg.reshape(batch, ref_h * ref_w)
    out = _masked_topk(corr3, fg_flat)
    return out.reshape(batch, 2 * KEEP, cur_h, cur_w)


# speedup vs baseline: 1.4685x; 1.4685x over previous
import functools

import jax
import jax.numpy as jnp
import numpy as np
from jax import lax
from jax.experimental import pallas as pl
from jax.experimental.pallas import tpu as pltpu
from jax.experimental.pallas import tpu_sc as plsc

KEEP = 32
NW = 32
R_BLK = 32
ROWS_PER_W = 512
N_BLK = ROWS_PER_W // R_BLK


def _resize_matrix(out_n: int, in_n: int) -> np.ndarray:
    scale = out_n / in_n
    kernel_scale = max(1.0, 1.0 / scale)
    sample_f = (np.arange(out_n) + 0.5) / scale - 0.5
    x = np.abs(sample_f[:, None] - np.arange(in_n)[None, :]) / kernel_scale
    a = np.maximum(0.0, 1.0 - x)
    a = a / a.sum(axis=1, keepdims=True)
    return a.astype(np.float32)


_A = _resize_matrix(32, 512)


MB = 4


def _mask_body(a_ref, at_ref, m_ref, o_ref):
    for i in range(MB):
        t = jnp.dot(a_ref[...], m_ref[i], preferred_element_type=jnp.float32,
                    precision=jax.lax.Precision.HIGHEST)
        s = jnp.dot(t, at_ref[...], preferred_element_type=jnp.float32,
                    precision=jax.lax.Precision.HIGHEST)
        o_ref[i] = (s > 0.5).astype(jnp.float32)


def _compute_fg(ref_mask_sq):
    b = ref_mask_sq.shape[0]
    a = jnp.asarray(_A)
    return pl.pallas_call(
        _mask_body,
        grid=(b // MB,),
        in_specs=[
            pl.BlockSpec((32, 512), lambda i: (0, 0)),
            pl.BlockSpec((512, 32), lambda i: (0, 0)),
            pl.BlockSpec((MB, 512, 512), lambda i: (i, 0, 0)),
        ],
        out_specs=pl.BlockSpec((MB, 32, 32), lambda i: (i, 0, 0)),
        out_shape=jax.ShapeDtypeStruct((b, 32, 32), jnp.float32),
    )(a, a.T, ref_mask_sq)


def _sort_a(x):
    return plsc.sort_key_val(x, x)[0]


def _sort_d(x):
    return plsc.sort_key_val(x, x, descending=True)[0]


def _update(r0d, r1d, c0, c1):
    c0s = _sort_a(c0)
    c1d = _sort_d(c1)
    lo = jnp.minimum(c0s, c1d)
    hi = jnp.maximum(c0s, c1d)
    s0 = _sort_a(lo)
    s1 = _sort_a(hi)
    h0 = jnp.maximum(s0, r1d)
    h1 = jnp.maximum(s1, r0d)
    lo2 = jnp.minimum(h0, h1)
    hi2 = jnp.maximum(h0, h1)
    return _sort_d(lo2), _sort_d(hi2)


def _merge32(r0d, r1d, q0d, q1d):
    s0 = jnp.flip(q0d, 0)
    s1 = jnp.flip(q1d, 0)
    h0 = jnp.maximum(s0, r1d)
    h1 = jnp.maximum(s1, r0d)
    lo2 = jnp.minimum(h0, h1)
    hi2 = jnp.maximum(h0, h1)
    return _sort_d(lo2), _sort_d(hi2)


def _topk_body(corr_hbm, fg_hbm, out_hbm, mask_v, buf, res, candf, candg,
               sem0, sem1):
    w = lax.axis_index("s") * 2 + lax.axis_index("c")
    b = w // 2
    half = w % 2
    row0 = half * ROWS_PER_W

    ks = lax.iota(jnp.int32, 16)
    neg = jnp.full((16,), -jnp.inf, jnp.float32)

    def start(gb, slot, sem):
        pltpu.make_async_copy(
            corr_hbm.at[b, pl.ds(row0 + gb * R_BLK, R_BLK), :],
            buf.at[slot], sem).start()

    def wait(slot, sem):
        pltpu.make_async_copy(
            corr_hbm.at[b, pl.ds(row0, R_BLK), :], buf.at[slot], sem).wait()

    N_STATIC = 4

    def consume(cand, offs):
        def body(j, st):
            base = j * 32
            c0 = cand[pl.ds(base, 16)]
            c1 = cand[pl.ds(base + 16, 16)]
            j2 = jnp.full((16,), 2 * j, jnp.int32)
            c0 = jnp.where(j2 < offs, c0, neg)
            c1 = jnp.where(j2 + 1 < offs, c1, neg)
            return _update(st[0], st[1], c0, c1)

        trips = (jnp.max(offs) + 1) // 2
        return lax.fori_loop(0, trips, body, (neg, neg))

    def do_rows(gb, slot):
        def row_body(r, _):
            p = gb * R_BLK + r
            fm1, fm2, gm1, gm2 = neg, neg, neg, neg
            for k in range(64):
                off = k * 16
                v = buf[slot, r, pl.ds(off, 16)]
                m = mask_v[pl.ds(off, 16)]
                a = v * m
                d = v - a
                lo = jnp.minimum(a, fm1)
                fm1 = jnp.maximum(a, fm1)
                fm2 = jnp.maximum(fm2, lo)
                lo = jnp.minimum(d, gm1)
                gm1 = jnp.maximum(d, gm1)
                gm2 = jnp.maximum(gm2, lo)
            tfv = jnp.full((16,), jnp.min(fm2), jnp.float32)
            tgv = jnp.full((16,), jnp.min(gm2), jnp.float32)
            idxf = ks
            idxg = ks
            for k in range(64):
                off = k * 16
                v = buf[slot, r, pl.ds(off, 16)]
                m = mask_v[pl.ds(off, 16)]
                a = v * m
                d = v - a
                sf = a >= tfv
                sg = d >= tgv
                plsc.store_scatter(candf, [idxf], a, mask=sf)
                plsc.store_scatter(candg, [idxg], d, mask=sg)
                idxf = idxf + (sf.astype(jnp.int32) << 4)
                idxg = idxg + (sg.astype(jnp.int32) << 4)
            f0, f1 = consume(candf, (idxf - ks) >> 4)
            g0, g1 = consume(candg, (idxg - ks) >> 4)
            pv = jnp.full((16,), p, jnp.int32)
            plsc.store_scatter(res, [ks, pv], g1)
            plsc.store_scatter(res, [ks + 16, pv], g0)
            plsc.store_scatter(res, [ks + 32, pv], f1)
            plsc.store_scatter(res, [ks + 48, pv], f0)
            return 0

        lax.fori_loop(0, R_BLK, row_body, 0)

    start(0, 0, sem0)
    pltpu.sync_copy(fg_hbm.at[b], mask_v)

    def block_body(g2, _):
        gb0 = 2 * g2
        wait(0, sem0)
        start(gb0 + 1, 1, sem1)
        do_rows(gb0, 0)
        wait(1, sem1)

        @pl.when(g2 < N_BLK // 2 - 1)
        def _():
            start(gb0 + 2, 0, sem0)

        do_rows(gb0 + 1, 1)
        return 0

    lax.fori_loop(0, N_BLK // 2, block_body, 0)
    pltpu.sync_copy(res, out_hbm.at[b, :, pl.ds(row0, ROWS_PER_W)])


def _masked_topk(corr3, fg_flat):
    mesh = plsc.VectorSubcoreMesh(core_axis_name="c", subcore_axis_name="s",
                                  num_cores=2, num_subcores=16)
    f = pl.kernel(
        _topk_body,
        out_type=jax.ShapeDtypeStruct((16, 2 * KEEP, 1024), jnp.float32),
        mesh=mesh,
        compiler_params=pltpu.CompilerParams(needs_layout_passes=False),
        scratch_types=[
            pltpu.VMEM((1024,), jnp.float32),
            pltpu.VMEM((2, R_BLK, 1024), jnp.float32),
            pltpu.VMEM((2 * KEEP, ROWS_PER_W), jnp.float32),
            pltpu.VMEM((1040,), jnp.float32),
            pltpu.VMEM((1040,), jnp.float32),
            pltpu.SemaphoreType.DMA,
            pltpu.SemaphoreType.DMA,
        ],
    )
    return f(corr3, fg_flat)


def kernel(corr_features, ref_mask):
    batch, cur_h, cur_w, ref_h, ref_w = corr_features.shape
    corr3 = corr_features.reshape(batch, cur_h * cur_w, ref_h * ref_w)
    fg = _compute_fg(ref_mask.reshape(batch, 512, 512))
    fg_flat = fg.reshape(batch, ref_h * ref_w)
    out = _masked_topk(corr3, fg_flat)
    return out.reshape(batch, 2 * KEEP, cur_h, cur_w)


# speedup vs baseline: 1.5480x; 1.0541x over previous
import functools

import jax
import jax.numpy as jnp
import numpy as np
from jax import lax
from jax.experimental import pallas as pl
from jax.experimental.pallas import tpu as pltpu
from jax.experimental.pallas import tpu_sc as plsc

KEEP = 32
NW = 32
R_BLK = 32
ROWS_PER_W = 512
N_BLK = ROWS_PER_W // R_BLK


def _resize_matrix(out_n: int, in_n: int) -> np.ndarray:
    scale = out_n / in_n
    kernel_scale = max(1.0, 1.0 / scale)
    sample_f = (np.arange(out_n) + 0.5) / scale - 0.5
    x = np.abs(sample_f[:, None] - np.arange(in_n)[None, :]) / kernel_scale
    a = np.maximum(0.0, 1.0 - x)
    a = a / a.sum(axis=1, keepdims=True)
    return a.astype(np.float32)


_A = _resize_matrix(32, 512)


MB = 4


def _mask_body(a_ref, at_ref, m_ref, o_ref):
    for i in range(MB):
        t = jnp.dot(a_ref[...], m_ref[i], preferred_element_type=jnp.float32,
                    precision=jax.lax.Precision.HIGHEST)
        s = jnp.dot(t, at_ref[...], preferred_element_type=jnp.float32,
                    precision=jax.lax.Precision.HIGHEST)
        o_ref[i] = (s > 0.5).astype(jnp.float32)


def _compute_fg(ref_mask_sq):
    b = ref_mask_sq.shape[0]
    a = jnp.asarray(_A)
    return pl.pallas_call(
        _mask_body,
        grid=(b // MB,),
        in_specs=[
            pl.BlockSpec((32, 512), lambda i: (0, 0)),
            pl.BlockSpec((512, 32), lambda i: (0, 0)),
            pl.BlockSpec((MB, 512, 512), lambda i: (i, 0, 0)),
        ],
        out_specs=pl.BlockSpec((MB, 32, 32), lambda i: (i, 0, 0)),
        out_shape=jax.ShapeDtypeStruct((b, 32, 32), jnp.float32),
    )(a, a.T, ref_mask_sq)


def _sort_a(x):
    return plsc.sort_key_val(x, x)[0]


def _sort_d(x):
    return plsc.sort_key_val(x, x, descending=True)[0]


def _update(r0d, r1d, c0, c1):
    c0s = _sort_a(c0)
    c1d = _sort_d(c1)
    lo = jnp.minimum(c0s, c1d)
    hi = jnp.maximum(c0s, c1d)
    s0 = _sort_a(lo)
    s1 = _sort_a(hi)
    h0 = jnp.maximum(s0, r1d)
    h1 = jnp.maximum(s1, r0d)
    lo2 = jnp.minimum(h0, h1)
    hi2 = jnp.maximum(h0, h1)
    return _sort_d(lo2), _sort_d(hi2)


def _merge32(r0d, r1d, q0d, q1d):
    s0 = jnp.flip(q0d, 0)
    s1 = jnp.flip(q1d, 0)
    h0 = jnp.maximum(s0, r1d)
    h1 = jnp.maximum(s1, r0d)
    lo2 = jnp.minimum(h0, h1)
    hi2 = jnp.maximum(h0, h1)
    return _sort_d(lo2), _sort_d(hi2)


def _topk_body(corr_hbm, fg_hbm, out_hbm, mask_v, buf, res, candf, candg,
               sem0, sem1):
    w = lax.axis_index("s") * 2 + lax.axis_index("c")
    b = w // 2
    half = w % 2
    row0 = half * ROWS_PER_W

    ks = lax.iota(jnp.int32, 16)
    neg = jnp.full((16,), -jnp.inf, jnp.float32)

    def start(gb, slot, sem):
        pltpu.make_async_copy(
            corr_hbm.at[b, pl.ds(row0 + gb * R_BLK, R_BLK), :],
            buf.at[slot], sem).start()

    def wait(slot, sem):
        pltpu.make_async_copy(
            corr_hbm.at[b, pl.ds(row0, R_BLK), :], buf.at[slot], sem).wait()

    def consume2(ca, offsa, cb, offsb):
        def body(j, st):
            a0, a1, b0, b1 = st
            base = j * 32
            ca0 = ca[pl.ds(base, 16)]
            ca1 = ca[pl.ds(base + 16, 16)]
            cb0 = cb[pl.ds(base, 16)]
            cb1 = cb[pl.ds(base + 16, 16)]
            j2 = jnp.full((16,), 2 * j, jnp.int32)
            ca0 = jnp.where(j2 < offsa, ca0, neg)
            ca1 = jnp.where(j2 + 1 < offsa, ca1, neg)
            cb0 = jnp.where(j2 < offsb, cb0, neg)
            cb1 = jnp.where(j2 + 1 < offsb, cb1, neg)
            a0, a1 = _update(a0, a1, ca0, ca1)
            b0, b1 = _update(b0, b1, cb0, cb1)
            return (a0, a1, b0, b1)

        trips = (jnp.maximum(jnp.max(offsa), jnp.max(offsb)) + 1) // 2
        return lax.fori_loop(0, trips, body, (neg, neg, neg, neg))

    def do_rows(gb, slot):
        def row_body(r, _):
            p = gb * R_BLK + r
            fm1, fm2, gm1, gm2 = neg, neg, neg, neg
            for k in range(64):
                off = k * 16
                v = buf[slot, r, pl.ds(off, 16)]
                m = mask_v[pl.ds(off, 16)]
                a = v * m
                d = v - a
                lo = jnp.minimum(a, fm1)
                fm1 = jnp.maximum(a, fm1)
                fm2 = jnp.maximum(fm2, lo)
                lo = jnp.minimum(d, gm1)
                gm1 = jnp.maximum(d, gm1)
                gm2 = jnp.maximum(gm2, lo)
            tfv = jnp.full((16,), jnp.min(fm2), jnp.float32)
            tgv = jnp.full((16,), jnp.min(gm2), jnp.float32)
            idxf = ks
            idxg = ks
            for k in range(64):
                off = k * 16
                v = buf[slot, r, pl.ds(off, 16)]
                m = mask_v[pl.ds(off, 16)]
                a = v * m
                d = v - a
                sf = a >= tfv
                sg = d >= tgv
                plsc.store_scatter(candf, [idxf], a, mask=sf)
                plsc.store_scatter(candg, [idxg], d, mask=sg)
                idxf = idxf + (sf.astype(jnp.int32) << 4)
                idxg = idxg + (sg.astype(jnp.int32) << 4)
            f0, f1, g0, g1 = consume2(candf, (idxf - ks) >> 4,
                                      candg, (idxg - ks) >> 4)
            pv = jnp.full((16,), p, jnp.int32)
            plsc.store_scatter(res, [ks, pv], g1)
            plsc.store_scatter(res, [ks + 16, pv], g0)
            plsc.store_scatter(res, [ks + 32, pv], f1)
            plsc.store_scatter(res, [ks + 48, pv], f0)
            return 0

        lax.fori_loop(0, R_BLK, row_body, 0)

    start(0, 0, sem0)
    pltpu.sync_copy(fg_hbm.at[b], mask_v)

    def block_body(g2, _):
        gb0 = 2 * g2
        wait(0, sem0)
        start(gb0 + 1, 1, sem1)
        do_rows(gb0, 0)
        wait(1, sem1)

        @pl.when(g2 < N_BLK // 2 - 1)
        def _():
            start(gb0 + 2, 0, sem0)

        do_rows(gb0 + 1, 1)
        return 0

    lax.fori_loop(0, N_BLK // 2, block_body, 0)
    pltpu.sync_copy(res, out_hbm.at[b, :, pl.ds(row0, ROWS_PER_W)])


def _masked_topk(corr3, fg_flat):
    mesh = plsc.VectorSubcoreMesh(core_axis_name="c", subcore_axis_name="s",
                                  num_cores=2, num_subcores=16)
    f = pl.kernel(
        _topk_body,
        out_type=jax.ShapeDtypeStruct((16, 2 * KEEP, 1024), jnp.float32),
        mesh=mesh,
        compiler_params=pltpu.CompilerParams(needs_layout_passes=False),
        scratch_types=[
            pltpu.VMEM((1024,), jnp.float32),
            pltpu.VMEM((2, R_BLK, 1024), jnp.float32),
            pltpu.VMEM((2 * KEEP, ROWS_PER_W), jnp.float32),
            pltpu.VMEM((1040,), jnp.float32),
            pltpu.VMEM((1040,), jnp.float32),
            pltpu.SemaphoreType.DMA,
            pltpu.SemaphoreType.DMA,
        ],
    )
    return f(corr3, fg_flat)


def kernel(corr_features, ref_mask):
    batch, cur_h, cur_w, ref_h, ref_w = corr_features.shape
    corr3 = corr_features.reshape(batch, cur_h * cur_w, ref_h * ref_w)
    fg = _compute_fg(ref_mask.reshape(batch, 512, 512))
    fg_flat = fg.reshape(batch, ref_h * ref_w)
    out = _masked_topk(corr3, fg_flat)
    return out.reshape(batch, 2 * KEEP, cur_h, cur_w)


# speedup vs baseline: 1.5524x; 1.0029x over previous
import functools

import jax
import jax.numpy as jnp
import numpy as np
from jax import lax
from jax.experimental import pallas as pl
from jax.experimental.pallas import tpu as pltpu
from jax.experimental.pallas import tpu_sc as plsc

KEEP = 32
NW = 32
R_BLK = 32
ROWS_PER_W = 512
N_BLK = ROWS_PER_W // R_BLK


def _resize_matrix(out_n: int, in_n: int) -> np.ndarray:
    scale = out_n / in_n
    kernel_scale = max(1.0, 1.0 / scale)
    sample_f = (np.arange(out_n) + 0.5) / scale - 0.5
    x = np.abs(sample_f[:, None] - np.arange(in_n)[None, :]) / kernel_scale
    a = np.maximum(0.0, 1.0 - x)
    a = a / a.sum(axis=1, keepdims=True)
    return a.astype(np.float32)


_A = _resize_matrix(32, 512)


MB = 4


def _mask_body(a_ref, at_ref, m_ref, o_ref):
    for i in range(MB):
        t = jnp.dot(a_ref[...], m_ref[i], preferred_element_type=jnp.float32,
                    precision=jax.lax.Precision.HIGHEST)
        s = jnp.dot(t, at_ref[...], preferred_element_type=jnp.float32,
                    precision=jax.lax.Precision.HIGHEST)
        o_ref[i] = (s > 0.5).astype(jnp.float32)


def _compute_fg(ref_mask_sq):
    b = ref_mask_sq.shape[0]
    a = jnp.asarray(_A)
    return pl.pallas_call(
        _mask_body,
        grid=(b // MB,),
        in_specs=[
            pl.BlockSpec((32, 512), lambda i: (0, 0)),
            pl.BlockSpec((512, 32), lambda i: (0, 0)),
            pl.BlockSpec((MB, 512, 512), lambda i: (i, 0, 0)),
        ],
        out_specs=pl.BlockSpec((MB, 32, 32), lambda i: (i, 0, 0)),
        out_shape=jax.ShapeDtypeStruct((b, 32, 32), jnp.float32),
    )(a, a.T, ref_mask_sq)


def _sort_a(x):
    return plsc.sort_key_val(x, x)[0]


def _sort_d(x):
    return plsc.sort_key_val(x, x, descending=True)[0]


def _update(r0d, r1d, c0, c1):
    c0s = _sort_a(c0)
    c1d = _sort_d(c1)
    lo = jnp.minimum(c0s, c1d)
    hi = jnp.maximum(c0s, c1d)
    s0 = _sort_a(lo)
    s1 = _sort_a(hi)
    h0 = jnp.maximum(s0, r1d)
    h1 = jnp.maximum(s1, r0d)
    lo2 = jnp.minimum(h0, h1)
    hi2 = jnp.maximum(h0, h1)
    return _sort_d(lo2), _sort_d(hi2)


def _merge32(r0d, r1d, q0d, q1d):
    s0 = jnp.flip(q0d, 0)
    s1 = jnp.flip(q1d, 0)
    h0 = jnp.maximum(s0, r1d)
    h1 = jnp.maximum(s1, r0d)
    lo2 = jnp.minimum(h0, h1)
    hi2 = jnp.maximum(h0, h1)
    return _sort_d(lo2), _sort_d(hi2)


def _topk_body(corr_hbm, fg_hbm, out_hbm, mask_v, buf, res, candf, candg,
               sem0, sem1):
    w = lax.axis_index("s") * 2 + lax.axis_index("c")
    b = w // 2
    half = w % 2
    row0 = half * ROWS_PER_W

    ks = lax.iota(jnp.int32, 16)
    neg = jnp.full((16,), -jnp.inf, jnp.float32)

    def start(gb, slot, sem):
        pltpu.make_async_copy(
            corr_hbm.at[b, pl.ds(row0 + gb * R_BLK, R_BLK), :],
            buf.at[slot], sem).start()

    def wait(slot, sem):
        pltpu.make_async_copy(
            corr_hbm.at[b, pl.ds(row0, R_BLK), :], buf.at[slot], sem).wait()

    def consume2(ca, offsa, cb, offsb):
        def upd_pair(j, sta, stb):
            base = j * 32
            ca0 = ca[pl.ds(base, 16)]
            ca1 = ca[pl.ds(base + 16, 16)]
            cb0 = cb[pl.ds(base, 16)]
            cb1 = cb[pl.ds(base + 16, 16)]
            j2 = jnp.full((16,), 2 * j, jnp.int32)
            ca0 = jnp.where(j2 < offsa, ca0, neg)
            ca1 = jnp.where(j2 + 1 < offsa, ca1, neg)
            cb0 = jnp.where(j2 < offsb, cb0, neg)
            cb1 = jnp.where(j2 + 1 < offsb, cb1, neg)
            return (_update(sta[0], sta[1], ca0, ca1),
                    _update(stb[0], stb[1], cb0, cb1))

        def body(i, st):
            sae, sbe, sao, sbo = st[:2], st[2:4], st[4:6], st[6:]
            sae, sbe = upd_pair(2 * i, sae, sbe)
            sao, sbo = upd_pair(2 * i + 1, sao, sbo)
            return sae + sbe + sao + sbo

        trips = (jnp.maximum(jnp.max(offsa), jnp.max(offsb)) + 1) // 2
        st = lax.fori_loop(0, (trips + 1) // 2, body,
                           (neg, neg, neg, neg, neg, neg, neg, neg))
        fa = _merge32(st[0], st[1], st[4], st[5])
        fb = _merge32(st[2], st[3], st[6], st[7])
        return fa + fb

    def do_rows(gb, slot):
        def row_body(r, _):
            p = gb * R_BLK + r
            fm1, fm2, gm1, gm2 = neg, neg, neg, neg
            for k in range(64):
                off = k * 16
                v = buf[slot, r, pl.ds(off, 16)]
                m = mask_v[pl.ds(off, 16)]
                a = v * m
                d = v - a
                lo = jnp.minimum(a, fm1)
                fm1 = jnp.maximum(a, fm1)
                fm2 = jnp.maximum(fm2, lo)
                lo = jnp.minimum(d, gm1)
                gm1 = jnp.maximum(d, gm1)
                gm2 = jnp.maximum(gm2, lo)
            tfv = jnp.full((16,), jnp.min(fm2), jnp.float32)
            tgv = jnp.full((16,), jnp.min(gm2), jnp.float32)
            idxf = ks
            idxg = ks
            for k in range(64):
                off = k * 16
                v = buf[slot, r, pl.ds(off, 16)]
                m = mask_v[pl.ds(off, 16)]
                a = v * m
                d = v - a
                sf = a >= tfv
                sg = d >= tgv
                plsc.store_scatter(candf, [idxf], a, mask=sf)
                plsc.store_scatter(candg, [idxg], d, mask=sg)
                idxf = idxf + (sf.astype(jnp.int32) << 4)
                idxg = idxg + (sg.astype(jnp.int32) << 4)
            f0, f1, g0, g1 = consume2(candf, (idxf - ks) >> 4,
                                      candg, (idxg - ks) >> 4)
            pv = jnp.full((16,), p, jnp.int32)
            plsc.store_scatter(res, [ks, pv], g1)
            plsc.store_scatter(res, [ks + 16, pv], g0)
            plsc.store_scatter(res, [ks + 32, pv], f1)
            plsc.store_scatter(res, [ks + 48, pv], f0)
            return 0

        lax.fori_loop(0, R_BLK, row_body, 0)

    start(0, 0, sem0)
    pltpu.sync_copy(fg_hbm.at[b], mask_v)

    def block_body(g2, _):
        gb0 = 2 * g2
        wait(0, sem0)
        start(gb0 + 1, 1, sem1)
        do_rows(gb0, 0)
        wait(1, sem1)

        @pl.when(g2 < N_BLK // 2 - 1)
        def _():
            start(gb0 + 2, 0, sem0)

        do_rows(gb0 + 1, 1)
        return 0

    lax.fori_loop(0, N_BLK // 2, block_body, 0)
    pltpu.sync_copy(res, out_hbm.at[b, :, pl.ds(row0, ROWS_PER_W)])


def _masked_topk(corr3, fg_flat):
    mesh = plsc.VectorSubcoreMesh(core_axis_name="c", subcore_axis_name="s",
                                  num_cores=2, num_subcores=16)
    f = pl.kernel(
        _topk_body,
        out_type=jax.ShapeDtypeStruct((16, 2 * KEEP, 1024), jnp.float32),
        mesh=mesh,
        compiler_params=pltpu.CompilerParams(needs_layout_passes=False),
        scratch_types=[
            pltpu.VMEM((1024,), jnp.float32),
            pltpu.VMEM((2, R_BLK, 1024), jnp.float32),
            pltpu.VMEM((2 * KEEP, ROWS_PER_W), jnp.float32),
            pltpu.VMEM((1040,), jnp.float32),
            pltpu.VMEM((1040,), jnp.float32),
            pltpu.SemaphoreType.DMA,
            pltpu.SemaphoreType.DMA,
        ],
    )
    return f(corr3, fg_flat)


def kernel(corr_features, ref_mask):
    batch, cur_h, cur_w, ref_h, ref_w = corr_features.shape
    corr3 = corr_features.reshape(batch, cur_h * cur_w, ref_h * ref_w)
    fg = _compute_fg(ref_mask.reshape(batch, 512, 512))
    fg_flat = fg.reshape(batch, ref_h * ref_w)
    out = _masked_topk(corr3, fg_flat)
    return out.reshape(batch, 2 * KEEP, cur_h, cur_w)
